# Initial kernel scaffold; baseline (speedup 1.0000x reference)
import jax
import jax.numpy as jnp
from jax.experimental import pallas as pl


def kernel(input, eidx):
    raise NotImplementedError("write your pallas kernel here")


# speedup vs baseline: 4.5150x; 4.5150x over previous
import functools

import jax
import jax.numpy as jnp
from jax import lax
from jax.experimental import pallas as pl
from jax.experimental.pallas import tpu as pltpu
from jax.experimental.pallas import tpu_sc as plsc

V = 10000
D = 256
E = 160000
NC = 2
NS = 16
HALF = D // NC
CHUNK = 128
CHUNKS = 79
E_PAD = NS * CHUNKS * CHUNK
V_PAD = 10112
ZROWS = V_PAD // NS

_mesh = plsc.VectorSubcoreMesh(
    core_axis_name="c", subcore_axis_name="s", num_cores=NC, num_subcores=NS
)


@functools.partial(
    pl.kernel,
    out_type=(
        jax.ShapeDtypeStruct((V_PAD, HALF), jnp.float32),
        jax.ShapeDtypeStruct((V_PAD, HALF), jnp.float32),
    ),
    mesh=_mesh,
    scratch_types=[
        pltpu.VMEM((CHUNKS, CHUNK), jnp.int32),
        pltpu.VMEM((CHUNKS, CHUNK), jnp.int32),
        pltpu.VMEM((CHUNK, HALF), jnp.float32),
        pltpu.VMEM_SHARED((V_PAD, HALF), jnp.float32),
        pltpu.SemaphoreType.DMA,
    ],
)
def _sc_scatter(x_lo, x_hi, s_hbm, t_hbm, z_hbm, out_lo, out_hi,
                s_v, t_v, buf, acc, sem):
    cid = lax.axis_index("c")
    tid = lax.axis_index("s")

    pltpu.sync_copy(s_hbm.at[tid], s_v)
    pltpu.sync_copy(t_hbm.at[tid], t_v)
    pltpu.sync_copy(z_hbm.at[pl.ds(tid * ZROWS, ZROWS)],
                    acc.at[pl.ds(tid * ZROWS, ZROWS)])
    plsc.subcore_barrier()

    def run_half(x_hbm):
        def step(j, carry):
            pltpu.async_copy(x_hbm.at[s_v.at[j]], buf, sem).wait()
            pltpu.sync_copy(buf, acc.at[t_v.at[j]], add=True)
            return carry
        lax.fori_loop(0, CHUNKS, step, 0)

    @pl.when(cid == 0)
    def _():
        run_half(x_lo)

    @pl.when(cid == 1)
    def _():
        run_half(x_hi)

    plsc.subcore_barrier()

    @pl.when(cid == 0)
    def _():
        pltpu.sync_copy(acc.at[pl.ds(tid * ZROWS, ZROWS)],
                        out_lo.at[pl.ds(tid * ZROWS, ZROWS)])

    @pl.when(cid == 1)
    def _():
        pltpu.sync_copy(acc.at[pl.ds(tid * ZROWS, ZROWS)],
                        out_hi.at[pl.ds(tid * ZROWS, ZROWS)])


def kernel(input, eidx):
    x_lo = input[:HALF].T
    x_hi = input[HALF:].T
    sidx = eidx[0].astype(jnp.int32)
    tidx = eidx[1].astype(jnp.int32)
    pad = E_PAD - E
    s_p = jnp.concatenate([sidx, jnp.zeros((pad,), jnp.int32)])
    t_p = jnp.concatenate([tidx, jnp.full((pad,), V, jnp.int32)])
    s_r = s_p.reshape(NS, CHUNKS, CHUNK)
    t_r = t_p.reshape(NS, CHUNKS, CHUNK)
    zeros = jnp.zeros((V_PAD, HALF), jnp.float32)
    out_lo, out_hi = _sc_scatter(x_lo, x_hi, s_r, t_r, zeros)
    return jnp.concatenate([out_lo[:V], out_hi[:V]], axis=1).T
